# Initial kernel scaffold; baseline (speedup 1.0000x reference)
import jax
import jax.numpy as jnp
from jax.experimental import pallas as pl
from jax.experimental.pallas import tpu as pltpu

_NQ = 4
_K = 1024
_D = 256
_TBLK = 512


def _rvq_block_kernel(z_ref, cb_ref, csq_ref, out_ref, loss_ref):
    x = z_ref[0]
    r = x.T
    acc = jnp.zeros_like(r)
    tq = r.shape[0]
    kiota = jax.lax.broadcasted_iota(jnp.int32, (tq, _K), 1)
    loss_part = jnp.float32(0.0)
    for q in range(_NQ):
        cb = cb_ref[q]
        dot = jax.lax.dot_general(
            r, cb, (((1,), (1,)), ((), ())),
            preferred_element_type=jnp.float32)
        rsq = jnp.sum(r * r, axis=1, keepdims=True)
        dist = (rsq - 2.0 * dot) + csq_ref[q]
        m = jnp.min(dist, axis=1, keepdims=True)
        idx = jnp.min(jnp.where(dist == m, kiota, _K), axis=1)
        quant = jnp.take_along_axis(
            cb, jax.lax.broadcast_in_dim(idx, (tq, _D), (0,)),
            axis=0, mode="promise_in_bounds")
        if q == _NQ - 1:
            d2 = quant - r
            loss_part = jnp.sum(d2 * d2)
        acc = acc + (r + (quant - r))
        r = r - quant
    out_ref[0] = acc.T
    loss_ref[0, 0, 0, 0] = loss_part


def kernel(z_e, codebooks):
    b, d, t = z_e.shape
    nq, k, _ = codebooks.shape
    nt = t // _TBLK
    csq = jnp.sum(codebooks ** 2, axis=-1).reshape(nq, 1, k)

    grid = (b, nt)
    z_q, loss_parts = pl.pallas_call(
        _rvq_block_kernel,
        grid=grid,
        in_specs=[
            pl.BlockSpec((1, d, _TBLK), lambda i, j: (i, 0, j)),
            pl.BlockSpec((nq, k, d), lambda i, j: (0, 0, 0)),
            pl.BlockSpec((nq, 1, k), lambda i, j: (0, 0, 0)),
        ],
        out_specs=[
            pl.BlockSpec((1, d, _TBLK), lambda i, j: (i, 0, j)),
            pl.BlockSpec((1, 1, 1, 1), lambda i, j: (i, j, 0, 0)),
        ],
        out_shape=[
            jax.ShapeDtypeStruct((b, d, t), jnp.float32),
            jax.ShapeDtypeStruct((b, nt, 1, 1), jnp.float32),
        ],
        compiler_params=pltpu.CompilerParams(
            dimension_semantics=("parallel", "parallel"),
        ),
    )(z_e, codebooks, csq)

    loss = jnp.sum(loss_parts) / jnp.float32(b * t * d)
    perplexity = jnp.float32(0.0)
    return (z_q, loss, perplexity)


# speedup vs baseline: 1.2205x; 1.2205x over previous
import jax
import jax.numpy as jnp
from jax.experimental import pallas as pl
from jax.experimental.pallas import tpu as pltpu

_NQ = 4
_K = 1024
_D = 256
_TBLK = 256
_LANES = 128


def _rvq_block_kernel(z_ref, cb_ref, cbt_ref, csq_ref, out_ref, loss_ref):
    r = z_ref[0]
    acc = jnp.zeros_like(r)
    tq = r.shape[1]
    kiota = jax.lax.broadcasted_iota(jnp.int32, (_K, tq), 0)
    loss_part = jnp.zeros((1, 1), jnp.float32)
    for q in range(_NQ):
        cb = cb_ref[q]
        dot = jax.lax.dot_general(
            cb, r, (((1,), (0,)), ((), ())),
            preferred_element_type=jnp.float32)
        rsq = jnp.sum(r * r, axis=0, keepdims=True)
        dist = (rsq - 2.0 * dot) + csq_ref[q]
        m = jnp.min(dist, axis=0, keepdims=True)
        idx = jnp.min(jnp.where(dist == m, kiota, _K), axis=0)
        lo = jax.lax.broadcast_in_dim(idx & (_LANES - 1), (_D, tq), (1,))
        hi = jax.lax.broadcast_in_dim(idx >> 7, (_D, tq), (1,))
        quant = jnp.zeros((_D, tq), jnp.float32)
        for c in range(_K // _LANES):
            g = jnp.take_along_axis(
                cbt_ref[q, :, c * _LANES:(c + 1) * _LANES], lo,
                axis=1, mode="promise_in_bounds")
            quant = jnp.where(hi == c, g, quant)
        if q == _NQ - 1:
            d2 = quant - r
            loss_part = jnp.sum(d2 * d2, keepdims=True).reshape(1, 1)
        acc = acc + (r + (quant - r))
        r = r - quant
    out_ref[0] = acc
    loss_ref[0, 0] = loss_part


def kernel(z_e, codebooks):
    b, d, t = z_e.shape
    nq, k, _ = codebooks.shape
    nt = t // _TBLK
    csq = jnp.sum(codebooks ** 2, axis=-1).reshape(nq, k, 1)
    cbt = jnp.transpose(codebooks, (0, 2, 1))

    grid = (b, nt)
    z_q, loss_parts = pl.pallas_call(
        _rvq_block_kernel,
        grid=grid,
        in_specs=[
            pl.BlockSpec((1, d, _TBLK), lambda i, j: (i, 0, j)),
            pl.BlockSpec((nq, k, d), lambda i, j: (0, 0, 0)),
            pl.BlockSpec((nq, d, k), lambda i, j: (0, 0, 0)),
            pl.BlockSpec((nq, k, 1), lambda i, j: (0, 0, 0)),
        ],
        out_specs=[
            pl.BlockSpec((1, d, _TBLK), lambda i, j: (i, 0, j)),
            pl.BlockSpec((1, 1, 1, 1), lambda i, j: (i, j, 0, 0)),
        ],
        out_shape=[
            jax.ShapeDtypeStruct((b, d, t), jnp.float32),
            jax.ShapeDtypeStruct((b, nt, 1, 1), jnp.float32),
        ],
        compiler_params=pltpu.CompilerParams(
            dimension_semantics=("parallel", "parallel"),
        ),
    )(z_e, codebooks, cbt, csq)

    loss = jnp.sum(loss_parts) / jnp.float32(b * t * d)
    perplexity = jnp.float32(0.0)
    return (z_q, loss, perplexity)


# speedup vs baseline: 1.4676x; 1.2024x over previous
import jax
import jax.numpy as jnp
from jax.experimental import pallas as pl
from jax.experimental.pallas import tpu as pltpu

_NQ = 4
_K = 1024
_D = 256
_TSUB = 256
_NSUB = 2
_TBLK = _TSUB * _NSUB
_LANES = 128
_KC = 256


def _quantize_step(r, q, cb2_ref, cbt_ref, csq_ref, kiota):
    tq = r.shape[1]
    rsq = jnp.sum(r * r, axis=0, keepdims=True)
    m = jnp.full((1, tq), jnp.inf, jnp.float32)
    idx = jnp.zeros((tq,), jnp.int32)
    for kc in range(_K // _KC):
        cbc = cb2_ref[q, kc * _KC:(kc + 1) * _KC]
        dotc = jax.lax.dot_general(
            cbc, r, (((1,), (0,)), ((), ())),
            preferred_element_type=jnp.float32)
        distc = (rsq - dotc) + csq_ref[q, kc * _KC:(kc + 1) * _KC]
        mc = jnp.min(distc, axis=0, keepdims=True)
        idxc = jnp.min(jnp.where(distc == mc, kiota, _KC), axis=0) + kc * _KC
        better = mc < m
        idx = jnp.where(better[0], idxc, idx)
        m = jnp.where(better, mc, m)
    lo = jax.lax.broadcast_in_dim(idx & (_LANES - 1), (_D, tq), (1,))
    hi = jax.lax.broadcast_in_dim(idx >> 7, (_D, tq), (1,))
    gs = [
        jnp.take_along_axis(
            cbt_ref[q, :, c * _LANES:(c + 1) * _LANES], lo,
            axis=1, mode="promise_in_bounds")
        for c in range(_K // _LANES)
    ]
    b0 = (hi & 1) == 1
    gs = [jnp.where(b0, gs[2 * i + 1], gs[2 * i]) for i in range(4)]
    b1 = (hi & 2) == 2
    gs = [jnp.where(b1, gs[2 * i + 1], gs[2 * i]) for i in range(2)]
    b2 = (hi & 4) == 4
    return jnp.where(b2, gs[1], gs[0])


def _rvq_block_kernel(z_ref, cb2_ref, cbt_ref, csq_ref, out_ref, loss_ref):
    kiota = jax.lax.broadcasted_iota(jnp.int32, (_KC, _TSUB), 0)
    rs = [z_ref[0, :, h * _TSUB:(h + 1) * _TSUB] for h in range(_NSUB)]
    accs = [jnp.zeros((_D, _TSUB), jnp.float32) for _ in range(_NSUB)]
    loss_part = jnp.zeros((1, 1), jnp.float32)
    for q in range(_NQ):
        for h in range(_NSUB):
            r = rs[h]
            quant = _quantize_step(r, q, cb2_ref, cbt_ref, csq_ref, kiota)
            accs[h] = accs[h] + (r + (quant - r))
            rs[h] = r - quant
    for h in range(_NSUB):
        out_ref[0, :, h * _TSUB:(h + 1) * _TSUB] = accs[h]
    for h in range(_NSUB):
        rn = rs[h]
        loss_part = loss_part + jnp.sum(rn * rn, keepdims=True).reshape(1, 1)
    loss_ref[0, 0] = loss_part


def kernel(z_e, codebooks):
    b, d, t = z_e.shape
    nq, k, _ = codebooks.shape
    nt = t // _TBLK
    csq = jnp.sum(codebooks ** 2, axis=-1).reshape(nq, k, 1)
    cbt = jnp.transpose(codebooks, (0, 2, 1))
    cb2 = codebooks * 2.0

    grid = (b, nt)
    z_q, loss_parts = pl.pallas_call(
        _rvq_block_kernel,
        grid=grid,
        in_specs=[
            pl.BlockSpec((1, d, _TBLK), lambda i, j: (i, 0, j)),
            pl.BlockSpec((nq, k, d), lambda i, j: (0, 0, 0)),
            pl.BlockSpec((nq, d, k), lambda i, j: (0, 0, 0)),
            pl.BlockSpec((nq, k, 1), lambda i, j: (0, 0, 0)),
        ],
        out_specs=[
            pl.BlockSpec((1, d, _TBLK), lambda i, j: (i, 0, j)),
            pl.BlockSpec((1, 1, 1, 1), lambda i, j: (i, j, 0, 0)),
        ],
        out_shape=[
            jax.ShapeDtypeStruct((b, d, t), jnp.float32),
            jax.ShapeDtypeStruct((b, nt, 1, 1), jnp.float32),
        ],
        compiler_params=pltpu.CompilerParams(
            dimension_semantics=("parallel", "parallel"),
        ),
    )(z_e, cb2, cbt, csq)

    loss = jnp.sum(loss_parts) / jnp.float32(b * t * d)
    perplexity = jnp.float32(0.0)
    return (z_q, loss, perplexity)


# speedup vs baseline: 2.1607x; 1.4723x over previous
import jax
import jax.numpy as jnp
from jax.experimental import pallas as pl
from jax.experimental.pallas import tpu as pltpu

_NQ = 4
_K = 1024
_D = 256
_TSUB = 128
_NSUB = 8
_TBLK = _TSUB * _NSUB
_LANES = 128
_KC = 256


def _quantize_step(r, q, cb2_ref, cbt_ref, csq_ref, kiota):
    tq = r.shape[1]
    rsq = jnp.sum(r * r, axis=0, keepdims=True)
    m = jnp.full((1, tq), jnp.inf, jnp.float32)
    idx = jnp.zeros((tq,), jnp.int32)
    for kc in range(_K // _KC):
        cbc = cb2_ref[q, kc * _KC:(kc + 1) * _KC]
        dotc = jax.lax.dot_general(
            cbc, r, (((1,), (0,)), ((), ())),
            preferred_element_type=jnp.float32)
        distc = (rsq - dotc) + csq_ref[q, kc * _KC:(kc + 1) * _KC]
        mc = jnp.min(distc, axis=0, keepdims=True)
        idxc = jnp.min(jnp.where(distc == mc, kiota, _KC), axis=0) + kc * _KC
        better = mc < m
        idx = jnp.where(better[0], idxc, idx)
        m = jnp.where(better, mc, m)
    lo = jax.lax.broadcast_in_dim(idx & (_LANES - 1), (_D, tq), (1,))
    hi = jax.lax.broadcast_in_dim(idx >> 7, (_D, tq), (1,))
    gs = [
        jnp.take_along_axis(
            cbt_ref[q, :, c * _LANES:(c + 1) * _LANES], lo,
            axis=1, mode="promise_in_bounds")
        for c in range(_K // _LANES)
    ]
    b0 = (hi & 1) == 1
    gs = [jnp.where(b0, gs[2 * i + 1], gs[2 * i]) for i in range(4)]
    b1 = (hi & 2) == 2
    gs = [jnp.where(b1, gs[2 * i + 1], gs[2 * i]) for i in range(2)]
    b2 = (hi & 4) == 4
    return jnp.where(b2, gs[1], gs[0])


def _rvq_block_kernel(z_ref, cb2_ref, cbt_ref, csq_ref, out_ref, loss_ref):
    kiota = jax.lax.broadcasted_iota(jnp.int32, (_KC, _TSUB), 0)
    rs = [z_ref[0, :, h * _TSUB:(h + 1) * _TSUB] for h in range(_NSUB)]
    accs = [jnp.zeros((_D, _TSUB), jnp.float32) for _ in range(_NSUB)]
    loss_part = jnp.zeros((1, 1), jnp.float32)
    for q in range(_NQ):
        for h in range(_NSUB):
            r = rs[h]
            quant = _quantize_step(r, q, cb2_ref, cbt_ref, csq_ref, kiota)
            accs[h] = accs[h] + (r + (quant - r))
            rs[h] = r - quant
    for h in range(_NSUB):
        out_ref[0, :, h * _TSUB:(h + 1) * _TSUB] = accs[h]
    for h in range(_NSUB):
        rn = rs[h]
        loss_part = loss_part + jnp.sum(rn * rn, keepdims=True).reshape(1, 1)
    loss_ref[0, 0] = loss_part


def kernel(z_e, codebooks):
    b, d, t = z_e.shape
    nq, k, _ = codebooks.shape
    nt = t // _TBLK
    csq = jnp.sum(codebooks ** 2, axis=-1).reshape(nq, k, 1)
    cbt = jnp.transpose(codebooks, (0, 2, 1))
    cb2 = codebooks * 2.0

    grid = (b, nt)
    z_q, loss_parts = pl.pallas_call(
        _rvq_block_kernel,
        grid=grid,
        in_specs=[
            pl.BlockSpec((1, d, _TBLK), lambda i, j: (i, 0, j)),
            pl.BlockSpec((nq, k, d), lambda i, j: (0, 0, 0)),
            pl.BlockSpec((nq, d, k), lambda i, j: (0, 0, 0)),
            pl.BlockSpec((nq, k, 1), lambda i, j: (0, 0, 0)),
        ],
        out_specs=[
            pl.BlockSpec((1, d, _TBLK), lambda i, j: (i, 0, j)),
            pl.BlockSpec((1, 1, 1, 1), lambda i, j: (i, j, 0, 0)),
        ],
        out_shape=[
            jax.ShapeDtypeStruct((b, d, t), jnp.float32),
            jax.ShapeDtypeStruct((b, nt, 1, 1), jnp.float32),
        ],
        compiler_params=pltpu.CompilerParams(
            dimension_semantics=("parallel", "parallel"),
        ),
    )(z_e, cb2, cbt, csq)

    loss = jnp.sum(loss_parts) / jnp.float32(b * t * d)
    perplexity = jnp.float32(0.0)
    return (z_q, loss, perplexity)


# speedup vs baseline: 2.4815x; 1.1485x over previous
import jax
import jax.numpy as jnp
from jax.experimental import pallas as pl
from jax.experimental.pallas import tpu as pltpu

_NQ = 4
_K = 1024
_D = 256
_TSUB = 128
_NSUB = 8
_TBLK = _TSUB * _NSUB
_LANES = 128
_KC = 1024


def _argmin_step(r, q, cb2_ref, csq_ref, kiota):
    tq = r.shape[1]
    rsq = jnp.sum(r * r, axis=0, keepdims=True)
    m = jnp.full((1, tq), jnp.inf, jnp.float32)
    idx = jnp.zeros((tq,), jnp.int32)
    for kc in range(_K // _KC):
        cbc = cb2_ref[q, kc * _KC:(kc + 1) * _KC]
        dotc = jax.lax.dot_general(
            cbc, r, (((1,), (0,)), ((), ())),
            preferred_element_type=jnp.float32)
        distc = (rsq - dotc) + csq_ref[q, kc * _KC:(kc + 1) * _KC]
        mc = jnp.min(distc, axis=0, keepdims=True)
        idxc = jnp.min(jnp.where(distc == mc, kiota, _KC), axis=0) + kc * _KC
        better = mc < m
        idx = jnp.where(better[0], idxc, idx)
        m = jnp.where(better, mc, m)
    return idx


def _rvq_block_kernel(z_ref, cb2_ref, cbt_ref, csq_ref, out_ref, loss_ref):
    kiota = jax.lax.broadcasted_iota(jnp.int32, (_KC, _TSUB), 0)
    rs = [z_ref[0, :, h * _TSUB:(h + 1) * _TSUB] for h in range(_NSUB)]
    accs = [jnp.zeros((_D, _TSUB), jnp.float32) for _ in range(_NSUB)]
    loss_part = jnp.zeros((1, 1), jnp.float32)
    for q in range(_NQ):
        for h in range(_NSUB):
            r = rs[h]
            idx = _argmin_step(r, q, cb2_ref, csq_ref, kiota)
            lo = jax.lax.broadcast_in_dim(idx & (_LANES - 1), (_D, _TSUB), (1,))
            hi = jax.lax.broadcast_in_dim(idx >> 7, (_D, _TSUB), (1,))
            gs = [
                jnp.take_along_axis(
                    cbt_ref[q, :, c * _LANES:(c + 1) * _LANES], lo,
                    axis=1, mode="promise_in_bounds")
                for c in range(_K // _LANES)
            ]
            b0 = (hi & 1) == 1
            gs = [jnp.where(b0, gs[2 * i + 1], gs[2 * i]) for i in range(4)]
            b1 = (hi & 2) == 2
            gs = [jnp.where(b1, gs[2 * i + 1], gs[2 * i]) for i in range(2)]
            b2 = (hi & 4) == 4
            quant = jnp.where(b2, gs[1], gs[0])
            accs[h] = accs[h] + (r + (quant - r))
            rs[h] = r - quant
    for h in range(_NSUB):
        out_ref[0, :, h * _TSUB:(h + 1) * _TSUB] = accs[h]
    for h in range(_NSUB):
        rn = rs[h]
        loss_part = loss_part + jnp.sum(rn * rn, keepdims=True).reshape(1, 1)
    loss_ref[0, 0] = loss_part


def kernel(z_e, codebooks):
    b, d, t = z_e.shape
    nq, k, _ = codebooks.shape
    nt = t // _TBLK
    csq = jnp.sum(codebooks ** 2, axis=-1).reshape(nq, k, 1)
    cbt = jnp.transpose(codebooks, (0, 2, 1))
    cb2 = codebooks * 2.0

    grid = (b, nt)
    z_q, loss_parts = pl.pallas_call(
        _rvq_block_kernel,
        grid=grid,
        in_specs=[
            pl.BlockSpec((1, d, _TBLK), lambda i, j: (i, 0, j)),
            pl.BlockSpec((nq, k, d), lambda i, j: (0, 0, 0)),
            pl.BlockSpec((nq, d, k), lambda i, j: (0, 0, 0)),
            pl.BlockSpec((nq, k, 1), lambda i, j: (0, 0, 0)),
        ],
        out_specs=[
            pl.BlockSpec((1, d, _TBLK), lambda i, j: (i, 0, j)),
            pl.BlockSpec((1, 1, 1, 1), lambda i, j: (i, j, 0, 0)),
        ],
        out_shape=[
            jax.ShapeDtypeStruct((b, d, t), jnp.float32),
            jax.ShapeDtypeStruct((b, nt, 1, 1), jnp.float32),
        ],
        compiler_params=pltpu.CompilerParams(
            dimension_semantics=("parallel", "parallel"),
        ),
    )(z_e, cb2, cbt, csq)

    loss = jnp.sum(loss_parts) / jnp.float32(b * t * d)
    perplexity = jnp.float32(0.0)
    return (z_q, loss, perplexity)
